# baseline (speedup 1.0000x reference)
import functools

import jax
import jax.numpy as jnp
from jax import lax
from jax.experimental import pallas as pl
from jax.experimental.pallas import tpu as pltpu
from jax.experimental.pallas import tpu_sc as plsc

NC = 2
NS = 16
NW = NC * NS
K = 128


def _zero_rows(ref, nrows, width):
    def body(i, _):
        for j in range(width // 16):
            ref[i, pl.ds(j * 16, 16)] = jnp.zeros((16,), jnp.float32)
        return 0
    lax.fori_loop(0, nrows, body, 0)


def _make_deg_kernel(NP, CH):
    RPT = NP // NS
    Fd = 16
    mesh = plsc.VectorSubcoreMesh(core_axis_name="c", subcore_axis_name="s")

    @functools.partial(
        pl.kernel,
        out_type=jax.ShapeDtypeStruct((NC, NP, Fd), jnp.float32),
        mesh=mesh,
        scratch_types=[
            pltpu.VMEM((CH, K), jnp.int32),
            pltpu.VMEM((K, Fd), jnp.float32),
            pltpu.VMEM((RPT, Fd), jnp.float32),
            pltpu.VMEM_SHARED((NP, Fd), jnp.float32),
            pltpu.SemaphoreType.DMA,
        ],
        compiler_params=pltpu.CompilerParams(use_tc_tiling_on_sc=False),
    )
    def deg_kernel(dst_hbm, out_hbm, dst_v, ones_v, zbuf_v, acc_sh, sem_s):
        c = lax.axis_index("c")
        s = lax.axis_index("s")
        wid = s * NC + c

        pltpu.sync_copy(dst_hbm.at[pl.ds(wid * CH, CH)], dst_v)

        def fill_ones(i, _):
            ones_v[i, pl.ds(0, 16)] = jnp.full((16,), 1.0, jnp.float32)
            return 0
        lax.fori_loop(0, K, fill_ones, 0)
        _zero_rows(zbuf_v, RPT, Fd)

        pltpu.sync_copy(zbuf_v, acc_sh.at[pl.ds(s * RPT, RPT)])
        plsc.subcore_barrier()

        def step(j, _):
            pltpu.async_copy(ones_v, acc_sh.at[dst_v.at[j]], sem_s, add=True)

            @pl.when(j >= 8)
            def _():
                pltpu.make_async_copy(
                    ones_v, acc_sh.at[dst_v.at[j]], sem_s).wait()
            return 0
        lax.fori_loop(0, CH, step, 0)

        def drain(j, _):
            pltpu.make_async_copy(ones_v, acc_sh.at[dst_v.at[j]], sem_s).wait()
            return 0
        lax.fori_loop(0, min(CH, 8), drain, 0)

        plsc.subcore_barrier()
        pltpu.sync_copy(acc_sh.at[pl.ds(s * RPT, RPT)], zbuf_v)
        pltpu.sync_copy(zbuf_v, out_hbm.at[c, pl.ds(s * RPT, RPT)])

    return deg_kernel


PF = 8


def _make_agg_kernel(NP, CH, F):
    RPT = NP // NS
    mesh = plsc.VectorSubcoreMesh(core_axis_name="c", subcore_axis_name="s")
    assert CH % PF == 0 and CH >= PF

    @functools.partial(
        pl.kernel,
        out_type=jax.ShapeDtypeStruct((NC, NP, F), jnp.float32),
        mesh=mesh,
        scratch_types=[
            pltpu.VMEM((CH, K), jnp.int32),
            pltpu.VMEM((CH, K), jnp.int32),
            pltpu.VMEM((PF, K, F), jnp.float32),
            pltpu.VMEM((RPT, F), jnp.float32),
            pltpu.VMEM_SHARED((NP, F), jnp.float32),
        ] + [pltpu.SemaphoreType.DMA] * PF,
        compiler_params=pltpu.CompilerParams(use_tc_tiling_on_sc=False),
    )
    def agg_kernel(g_hbm, src_hbm, dst_hbm, out_hbm,
                   src_v, dst_v, rows_v, zbuf_v, acc_sh, *sem_g):
        c = lax.axis_index("c")
        s = lax.axis_index("s")
        wid = s * NC + c

        pltpu.sync_copy(src_hbm.at[pl.ds(wid * CH, CH)], src_v)
        pltpu.sync_copy(dst_hbm.at[pl.ds(wid * CH, CH)], dst_v)
        _zero_rows(zbuf_v, RPT, F)

        pltpu.sync_copy(zbuf_v, acc_sh.at[pl.ds(s * RPT, RPT)])
        plsc.subcore_barrier()

        def gather(ck, b):
            return pltpu.make_async_copy(
                g_hbm.at[src_v.at[ck]], rows_v.at[b], sem_g[b])

        for ck in range(PF):
            gather(ck, ck).start()

        def group(j, _):
            c0 = j * PF
            for b in range(PF):
                ck = c0 + b
                gather(ck, b).wait()
                pltpu.sync_copy(rows_v.at[b], acc_sh.at[dst_v.at[ck]],
                                add=True)

                @pl.when(ck + PF < CH)
                def _():
                    gather(ck + PF, b).start()
            return 0
        lax.fori_loop(0, CH // PF, group, 0)

        plsc.subcore_barrier()
        pltpu.sync_copy(acc_sh.at[pl.ds(s * RPT, RPT)], zbuf_v)
        pltpu.sync_copy(zbuf_v, out_hbm.at[c, pl.ds(s * RPT, RPT)])

    return agg_kernel


def _dinv_block(dp):
    deg = dp[0, :, 0:1] + dp[1, :, 0:1] + 1.0
    return lax.rsqrt(jnp.maximum(deg, 1.0))


def _tc_first_body(x_ref, gl_ref, w_ref, dp_ref, o_ref):
    dinv = _dinv_block(dp_ref[...])
    h0 = jnp.dot(x_ref[...], gl_ref[...], preferred_element_type=jnp.float32)
    o_ref[...] = jnp.dot(h0, w_ref[...], preferred_element_type=jnp.float32) * dinv


def _tc_mid_body(ap_ref, g_ref, b_ref, w_ref, dp_ref, o_ref):
    dinv = _dinv_block(dp_ref[...])
    pre = ap_ref[0] + ap_ref[1] + g_ref[...]
    h = jnp.maximum(pre * dinv + b_ref[...], 0.0)
    o_ref[...] = jnp.dot(h, w_ref[...], preferred_element_type=jnp.float32) * dinv


def _tc_final_body(ap_ref, g_ref, b_ref, dp_ref, o_ref):
    dinv = _dinv_block(dp_ref[...])
    h = (ap_ref[0] + ap_ref[1] + g_ref[...]) * dinv + b_ref[...]
    m = jnp.max(h, axis=1, keepdims=True)
    z = h - m
    lse = jnp.log(jnp.sum(jnp.exp(z), axis=1, keepdims=True))
    o_ref[...] = z - lse


_BN = 1024


def _tc_first(x, glove, W1, degp):
    N, D = x.shape
    H = W1.shape[1]
    grid = (pl.cdiv(N, _BN),)
    return pl.pallas_call(
        _tc_first_body,
        grid=grid,
        in_specs=[
            pl.BlockSpec((_BN, D), lambda i: (i, 0)),
            pl.BlockSpec((D, D), lambda i: (0, 0)),
            pl.BlockSpec((D, H), lambda i: (0, 0)),
            pl.BlockSpec((2, _BN, 16), lambda i: (0, i, 0)),
        ],
        out_specs=pl.BlockSpec((_BN, H), lambda i: (i, 0)),
        out_shape=jax.ShapeDtypeStruct((N, H), jnp.float32),
    )(x, glove, W1, degp)


def _tc_mid(ap, g, b, W, degp):
    N, F = g.shape
    H2 = W.shape[1]
    grid = (pl.cdiv(N, _BN),)
    return pl.pallas_call(
        _tc_mid_body,
        grid=grid,
        in_specs=[
            pl.BlockSpec((2, _BN, F), lambda i: (0, i, 0)),
            pl.BlockSpec((_BN, F), lambda i: (i, 0)),
            pl.BlockSpec((1, F), lambda i: (0, 0)),
            pl.BlockSpec((F, H2), lambda i: (0, 0)),
            pl.BlockSpec((2, _BN, 16), lambda i: (0, i, 0)),
        ],
        out_specs=pl.BlockSpec((_BN, H2), lambda i: (i, 0)),
        out_shape=jax.ShapeDtypeStruct((N, H2), jnp.float32),
    )(ap, g, b, W, degp)


def _tc_final(ap, g, b, degp):
    N, F = g.shape
    grid = (pl.cdiv(N, _BN),)
    return pl.pallas_call(
        _tc_final_body,
        grid=grid,
        in_specs=[
            pl.BlockSpec((2, _BN, F), lambda i: (0, i, 0)),
            pl.BlockSpec((_BN, F), lambda i: (i, 0)),
            pl.BlockSpec((1, F), lambda i: (0, 0)),
            pl.BlockSpec((2, _BN, 16), lambda i: (0, i, 0)),
        ],
        out_specs=pl.BlockSpec((_BN, F), lambda i: (i, 0)),
        out_shape=jax.ShapeDtypeStruct((N, F), jnp.float32),
    )(ap, g, b, degp)


def kernel(x, edge_index, glove, W1, b1, W2, b2, W3, b3):
    N, D = x.shape
    E = edge_index.shape[1]
    H = W1.shape[1]
    C = W3.shape[1]

    NP = (-(-(N + 1) // 128)) * 128
    CH = -(-E // (NW * K))
    CH = (-(-CH // 8)) * 8
    EP = NW * CH * K
    pad = EP - E

    src = jnp.concatenate(
        [edge_index[0], jnp.zeros((pad,), jnp.int32)]).reshape(NW * CH, K)
    dst = jnp.concatenate(
        [edge_index[1], jnp.full((pad,), N, jnp.int32)]).reshape(NW * CH, K)

    deg_k = _make_deg_kernel(NP, CH)
    agg_h = _make_agg_kernel(NP, CH, H)
    agg_c = _make_agg_kernel(NP, CH, C)

    degp = deg_k(dst)
    g1 = _tc_first(x, glove, W1, degp)
    a1 = agg_h(g1, src, dst)
    g2 = _tc_mid(a1, g1, b1.reshape(1, H), W2, degp)
    a2 = agg_h(g2, src, dst)
    g3 = _tc_mid(a2, g2, b2.reshape(1, H), W3, degp)
    a3 = agg_c(g3, src, dst)
    return _tc_final(a3, g3, b3.reshape(1, C), degp)


# speedup vs baseline: 1.0080x; 1.0080x over previous
import functools

import jax
import jax.numpy as jnp
from jax import lax
from jax.experimental import pallas as pl
from jax.experimental.pallas import tpu as pltpu
from jax.experimental.pallas import tpu_sc as plsc

NC = 2
NS = 16
NW = NC * NS
K = 128


def _zero_rows(ref, nrows, width):
    def body(i, _):
        for j in range(width // 16):
            ref[i, pl.ds(j * 16, 16)] = jnp.zeros((16,), jnp.float32)
        return 0
    lax.fori_loop(0, nrows, body, 0)


def _make_deg_kernel(NP, CH):
    RPT = NP // NS
    Fd = 16
    mesh = plsc.VectorSubcoreMesh(core_axis_name="c", subcore_axis_name="s")

    @functools.partial(
        pl.kernel,
        out_type=jax.ShapeDtypeStruct((NC, NP, Fd), jnp.float32),
        mesh=mesh,
        scratch_types=[
            pltpu.VMEM((CH, K), jnp.int32),
            pltpu.VMEM((K, Fd), jnp.float32),
            pltpu.VMEM((RPT, Fd), jnp.float32),
            pltpu.VMEM_SHARED((NP, Fd), jnp.float32),
            pltpu.SemaphoreType.DMA,
        ],
        compiler_params=pltpu.CompilerParams(use_tc_tiling_on_sc=False),
    )
    def deg_kernel(dst_hbm, out_hbm, dst_v, ones_v, zbuf_v, acc_sh, sem_s):
        c = lax.axis_index("c")
        s = lax.axis_index("s")
        wid = s * NC + c

        pltpu.sync_copy(dst_hbm.at[pl.ds(wid * CH, CH)], dst_v)

        def fill_ones(i, _):
            ones_v[i, pl.ds(0, 16)] = jnp.full((16,), 1.0, jnp.float32)
            return 0
        lax.fori_loop(0, K, fill_ones, 0)
        _zero_rows(zbuf_v, RPT, Fd)

        pltpu.sync_copy(zbuf_v, acc_sh.at[pl.ds(s * RPT, RPT)])
        plsc.subcore_barrier()

        def step(j, _):
            pltpu.async_copy(ones_v, acc_sh.at[dst_v.at[j]], sem_s, add=True)

            @pl.when(j >= 8)
            def _():
                pltpu.make_async_copy(
                    ones_v, acc_sh.at[dst_v.at[j]], sem_s).wait()
            return 0
        lax.fori_loop(0, CH, step, 0)

        def drain(j, _):
            pltpu.make_async_copy(ones_v, acc_sh.at[dst_v.at[j]], sem_s).wait()
            return 0
        lax.fori_loop(0, min(CH, 8), drain, 0)

        plsc.subcore_barrier()
        pltpu.sync_copy(acc_sh.at[pl.ds(s * RPT, RPT)], zbuf_v)
        pltpu.sync_copy(zbuf_v, out_hbm.at[c, pl.ds(s * RPT, RPT)])

    return deg_kernel


PF = 4


def _make_agg_kernel(NP, CH, F):
    RPT = NP // NS
    mesh = plsc.VectorSubcoreMesh(core_axis_name="c", subcore_axis_name="s")
    assert CH % PF == 0 and CH >= PF

    NB = 2 * PF
    assert CH % NB == 0 and CH >= NB

    @functools.partial(
        pl.kernel,
        out_type=jax.ShapeDtypeStruct((NC, NP, F), jnp.float32),
        mesh=mesh,
        scratch_types=[
            pltpu.VMEM((CH, K), jnp.int32),
            pltpu.VMEM((CH, K), jnp.int32),
            pltpu.VMEM((NB, K, F), jnp.float32),
            pltpu.VMEM((RPT, F), jnp.float32),
            pltpu.VMEM_SHARED((NP, F), jnp.float32),
        ] + [pltpu.SemaphoreType.DMA] * (2 * NB),
        compiler_params=pltpu.CompilerParams(use_tc_tiling_on_sc=False),
    )
    def agg_kernel(g_hbm, src_hbm, dst_hbm, out_hbm,
                   src_v, dst_v, rows_v, zbuf_v, acc_sh, *sems):
        sem_g = sems[:NB]
        sem_s = sems[NB:]
        c = lax.axis_index("c")
        s = lax.axis_index("s")
        wid = s * NC + c

        pltpu.sync_copy(src_hbm.at[pl.ds(wid * CH, CH)], src_v)
        pltpu.sync_copy(dst_hbm.at[pl.ds(wid * CH, CH)], dst_v)
        _zero_rows(zbuf_v, RPT, F)

        pltpu.sync_copy(zbuf_v, acc_sh.at[pl.ds(s * RPT, RPT)])
        plsc.subcore_barrier()

        def gather(ck, b):
            return pltpu.make_async_copy(
                g_hbm.at[src_v.at[ck]], rows_v.at[b], sem_g[b])

        def scatter(ck, b):
            return pltpu.make_async_copy(
                rows_v.at[b], acc_sh.at[dst_v.at[ck]], sem_s[b])

        for ck in range(PF):
            gather(ck, ck % NB).start()

        def group(j, _):
            c0 = j * NB
            for b in range(NB):
                ck = c0 + b
                gather(ck, b).wait()
                scatter(ck, b).start(add=True)
                pre = ck + PF
                bp = (b + PF) % NB

                @pl.when(pre < CH)
                def _():
                    @pl.when(pre >= NB)
                    def _():
                        scatter(pre - NB, bp).wait()
                    gather(pre, bp).start()
            return 0
        lax.fori_loop(0, CH // NB, group, 0)

        for b in range(NB):
            scatter(CH - NB + b, b).wait()

        plsc.subcore_barrier()
        pltpu.sync_copy(acc_sh.at[pl.ds(s * RPT, RPT)], zbuf_v)
        pltpu.sync_copy(zbuf_v, out_hbm.at[c, pl.ds(s * RPT, RPT)])

    return agg_kernel


def _dinv_block(dp):
    deg = dp[0, :, 0:1] + dp[1, :, 0:1] + 1.0
    return lax.rsqrt(jnp.maximum(deg, 1.0))


def _tc_first_body(x_ref, gl_ref, w_ref, dp_ref, o_ref):
    dinv = _dinv_block(dp_ref[...])
    h0 = jnp.dot(x_ref[...], gl_ref[...], preferred_element_type=jnp.float32)
    o_ref[...] = jnp.dot(h0, w_ref[...], preferred_element_type=jnp.float32) * dinv


def _tc_mid_body(ap_ref, g_ref, b_ref, w_ref, dp_ref, o_ref):
    dinv = _dinv_block(dp_ref[...])
    pre = ap_ref[0] + ap_ref[1] + g_ref[...]
    h = jnp.maximum(pre * dinv + b_ref[...], 0.0)
    o_ref[...] = jnp.dot(h, w_ref[...], preferred_element_type=jnp.float32) * dinv


def _tc_final_body(ap_ref, g_ref, b_ref, dp_ref, o_ref):
    dinv = _dinv_block(dp_ref[...])
    h = (ap_ref[0] + ap_ref[1] + g_ref[...]) * dinv + b_ref[...]
    m = jnp.max(h, axis=1, keepdims=True)
    z = h - m
    lse = jnp.log(jnp.sum(jnp.exp(z), axis=1, keepdims=True))
    o_ref[...] = z - lse


_BN = 1024


def _tc_first(x, glove, W1, degp):
    N, D = x.shape
    H = W1.shape[1]
    grid = (pl.cdiv(N, _BN),)
    return pl.pallas_call(
        _tc_first_body,
        grid=grid,
        in_specs=[
            pl.BlockSpec((_BN, D), lambda i: (i, 0)),
            pl.BlockSpec((D, D), lambda i: (0, 0)),
            pl.BlockSpec((D, H), lambda i: (0, 0)),
            pl.BlockSpec((2, _BN, 16), lambda i: (0, i, 0)),
        ],
        out_specs=pl.BlockSpec((_BN, H), lambda i: (i, 0)),
        out_shape=jax.ShapeDtypeStruct((N, H), jnp.float32),
    )(x, glove, W1, degp)


def _tc_mid(ap, g, b, W, degp):
    N, F = g.shape
    H2 = W.shape[1]
    grid = (pl.cdiv(N, _BN),)
    return pl.pallas_call(
        _tc_mid_body,
        grid=grid,
        in_specs=[
            pl.BlockSpec((2, _BN, F), lambda i: (0, i, 0)),
            pl.BlockSpec((_BN, F), lambda i: (i, 0)),
            pl.BlockSpec((1, F), lambda i: (0, 0)),
            pl.BlockSpec((F, H2), lambda i: (0, 0)),
            pl.BlockSpec((2, _BN, 16), lambda i: (0, i, 0)),
        ],
        out_specs=pl.BlockSpec((_BN, H2), lambda i: (i, 0)),
        out_shape=jax.ShapeDtypeStruct((N, H2), jnp.float32),
    )(ap, g, b, W, degp)


def _tc_final(ap, g, b, degp):
    N, F = g.shape
    grid = (pl.cdiv(N, _BN),)
    return pl.pallas_call(
        _tc_final_body,
        grid=grid,
        in_specs=[
            pl.BlockSpec((2, _BN, F), lambda i: (0, i, 0)),
            pl.BlockSpec((_BN, F), lambda i: (i, 0)),
            pl.BlockSpec((1, F), lambda i: (0, 0)),
            pl.BlockSpec((2, _BN, 16), lambda i: (0, i, 0)),
        ],
        out_specs=pl.BlockSpec((_BN, F), lambda i: (i, 0)),
        out_shape=jax.ShapeDtypeStruct((N, F), jnp.float32),
    )(ap, g, b, degp)


def kernel(x, edge_index, glove, W1, b1, W2, b2, W3, b3):
    N, D = x.shape
    E = edge_index.shape[1]
    H = W1.shape[1]
    C = W3.shape[1]

    NP = (-(-(N + 1) // 128)) * 128
    CH = -(-E // (NW * K))
    CH = (-(-CH // 8)) * 8
    EP = NW * CH * K
    pad = EP - E

    src = jnp.concatenate(
        [edge_index[0], jnp.zeros((pad,), jnp.int32)]).reshape(NW * CH, K)
    dst = jnp.concatenate(
        [edge_index[1], jnp.full((pad,), N, jnp.int32)]).reshape(NW * CH, K)

    deg_k = _make_deg_kernel(NP, CH)
    agg_h = _make_agg_kernel(NP, CH, H)
    agg_c = _make_agg_kernel(NP, CH, C)

    degp = deg_k(dst)
    g1 = _tc_first(x, glove, W1, degp)
    a1 = agg_h(g1, src, dst)
    g2 = _tc_mid(a1, g1, b1.reshape(1, H), W2, degp)
    a2 = agg_h(g2, src, dst)
    g3 = _tc_mid(a2, g2, b2.reshape(1, H), W3, degp)
    a3 = agg_c(g3, src, dst)
    return _tc_final(a3, g3, b3.reshape(1, C), degp)


# speedup vs baseline: 1.1877x; 1.1783x over previous
import functools

import jax
import jax.numpy as jnp
from jax import lax
from jax.experimental import pallas as pl
from jax.experimental.pallas import tpu as pltpu
from jax.experimental.pallas import tpu_sc as plsc

NC = 2
NS = 16
NW = NC * NS
K = 128


def _zero_rows(ref, nrows, width):
    def body(i, _):
        for j in range(width // 16):
            ref[i, pl.ds(j * 16, 16)] = jnp.zeros((16,), jnp.float32)
        return 0
    lax.fori_loop(0, nrows, body, 0)


def _chunk_base(c, s, CH0, CH1):
    base = lax.select(c == 0, s * CH0, 16 * CH0 + s * CH1)
    nch = lax.select(c == 0, CH0, CH1)
    return base, nch


def _make_deg_kernel(NP, CH0, CH1):
    RPT = NP // NS
    CHM = max(CH0, CH1)
    Fd = 16
    mesh = plsc.VectorSubcoreMesh(core_axis_name="c", subcore_axis_name="s")

    @functools.partial(
        pl.kernel,
        out_type=jax.ShapeDtypeStruct((NC, NP, Fd), jnp.float32),
        mesh=mesh,
        scratch_types=[
            pltpu.VMEM((CHM, K), jnp.int32),
            pltpu.VMEM((K, Fd), jnp.float32),
            pltpu.VMEM((RPT, Fd), jnp.float32),
            pltpu.VMEM_SHARED((NP, Fd), jnp.float32),
            pltpu.SemaphoreType.DMA,
        ],
        compiler_params=pltpu.CompilerParams(use_tc_tiling_on_sc=False),
    )
    def deg_kernel(dst_hbm, out_hbm, dst_v, ones_v, zbuf_v, acc_sh, sem_s):
        c = lax.axis_index("c")
        s = lax.axis_index("s")
        base, nch = _chunk_base(c, s, CH0, CH1)

        pltpu.sync_copy(dst_hbm.at[pl.ds(base, CHM)], dst_v)

        def fill_ones(i, _):
            ones_v[i, pl.ds(0, 16)] = jnp.full((16,), 1.0, jnp.float32)
            return 0
        lax.fori_loop(0, K, fill_ones, 0)
        _zero_rows(zbuf_v, RPT, Fd)

        pltpu.sync_copy(zbuf_v, acc_sh.at[pl.ds(s * RPT, RPT)])
        plsc.subcore_barrier()

        def step(j, _):
            pltpu.async_copy(ones_v, acc_sh.at[dst_v.at[j]], sem_s, add=True)

            @pl.when(j >= 8)
            def _():
                pltpu.make_async_copy(
                    ones_v, acc_sh.at[dst_v.at[j]], sem_s).wait()
            return 0
        lax.fori_loop(0, nch, step, 0)

        def drain(j, _):
            pltpu.make_async_copy(ones_v, acc_sh.at[dst_v.at[j]], sem_s).wait()
            return 0
        lax.fori_loop(0, 8, drain, 0)

        plsc.subcore_barrier()
        pltpu.sync_copy(acc_sh.at[pl.ds(s * RPT, RPT)], zbuf_v)
        pltpu.sync_copy(zbuf_v, out_hbm.at[c, pl.ds(s * RPT, RPT)])

    return deg_kernel


PF = 4


def _make_agg_kernel(NP, CH0, CH1, F):
    RPT = NP // NS
    CHM = max(CH0, CH1)
    mesh = plsc.VectorSubcoreMesh(core_axis_name="c", subcore_axis_name="s")

    NB = 2 * PF
    assert CH0 % NB == 0 and CH0 >= NB
    assert CH1 % NB == 0 and CH1 >= NB

    @functools.partial(
        pl.kernel,
        out_type=jax.ShapeDtypeStruct((NC, NP, F), jnp.float32),
        mesh=mesh,
        scratch_types=[
            pltpu.VMEM((CHM, K), jnp.int32),
            pltpu.VMEM((CHM, K), jnp.int32),
            pltpu.VMEM((NB, K, F), jnp.float32),
            pltpu.VMEM((RPT, F), jnp.float32),
            pltpu.VMEM_SHARED((NP, F), jnp.float32),
        ] + [pltpu.SemaphoreType.DMA] * (2 * NB),
        compiler_params=pltpu.CompilerParams(use_tc_tiling_on_sc=False),
    )
    def agg_kernel(g_hbm, src_hbm, dst_hbm, out_hbm,
                   src_v, dst_v, rows_v, zbuf_v, acc_sh, *sems):
        sem_g = sems[:NB]
        sem_s = sems[NB:]
        c = lax.axis_index("c")
        s = lax.axis_index("s")
        base, nch = _chunk_base(c, s, CH0, CH1)

        pltpu.sync_copy(src_hbm.at[pl.ds(base, CHM)], src_v)
        pltpu.sync_copy(dst_hbm.at[pl.ds(base, CHM)], dst_v)
        _zero_rows(zbuf_v, RPT, F)

        pltpu.sync_copy(zbuf_v, acc_sh.at[pl.ds(s * RPT, RPT)])
        plsc.subcore_barrier()

        def gather(ck, b):
            return pltpu.make_async_copy(
                g_hbm.at[src_v.at[ck]], rows_v.at[b], sem_g[b])

        def scatter(ck, b):
            return pltpu.make_async_copy(
                rows_v.at[b], acc_sh.at[dst_v.at[ck]], sem_s[b])

        for ck in range(PF):
            gather(ck, ck % NB).start()

        def group(j, _):
            c0 = j * NB
            for b in range(NB):
                ck = c0 + b
                gather(ck, b).wait()
                scatter(ck, b).start(add=True)
                pre = ck + PF
                bp = (b + PF) % NB

                @pl.when(pre < nch)
                def _():
                    @pl.when(pre >= NB)
                    def _():
                        scatter(pre - NB, bp).wait()
                    gather(pre, bp).start()
            return 0
        lax.fori_loop(0, nch // NB, group, 0)

        for b in range(NB):
            scatter(nch - NB + b, b).wait()

        plsc.subcore_barrier()
        pltpu.sync_copy(acc_sh.at[pl.ds(s * RPT, RPT)], zbuf_v)
        pltpu.sync_copy(zbuf_v, out_hbm.at[c, pl.ds(s * RPT, RPT)])

    return agg_kernel


def _dinv_block(dp):
    deg = dp[0, :, 0:1] + dp[1, :, 0:1] + 1.0
    return lax.rsqrt(jnp.maximum(deg, 1.0))


def _tc_first_body(x_ref, gl_ref, w_ref, dp_ref, o_ref):
    dinv = _dinv_block(dp_ref[...])
    h0 = jnp.dot(x_ref[...], gl_ref[...], preferred_element_type=jnp.float32)
    o_ref[...] = jnp.dot(h0, w_ref[...], preferred_element_type=jnp.float32) * dinv


def _tc_mid_body(ap_ref, g_ref, b_ref, w_ref, dp_ref, o_ref):
    dinv = _dinv_block(dp_ref[...])
    pre = ap_ref[0] + ap_ref[1] + g_ref[...]
    h = jnp.maximum(pre * dinv + b_ref[...], 0.0)
    o_ref[...] = jnp.dot(h, w_ref[...], preferred_element_type=jnp.float32) * dinv


def _tc_final_body(ap_ref, g_ref, b_ref, dp_ref, o_ref):
    dinv = _dinv_block(dp_ref[...])
    h = (ap_ref[0] + ap_ref[1] + g_ref[...]) * dinv + b_ref[...]
    m = jnp.max(h, axis=1, keepdims=True)
    z = h - m
    lse = jnp.log(jnp.sum(jnp.exp(z), axis=1, keepdims=True))
    o_ref[...] = z - lse


_BN = 1024


def _tc_first(x, glove, W1, degp):
    N, D = x.shape
    H = W1.shape[1]
    grid = (pl.cdiv(N, _BN),)
    return pl.pallas_call(
        _tc_first_body,
        grid=grid,
        in_specs=[
            pl.BlockSpec((_BN, D), lambda i: (i, 0)),
            pl.BlockSpec((D, D), lambda i: (0, 0)),
            pl.BlockSpec((D, H), lambda i: (0, 0)),
            pl.BlockSpec((2, _BN, 16), lambda i: (0, i, 0)),
        ],
        out_specs=pl.BlockSpec((_BN, H), lambda i: (i, 0)),
        out_shape=jax.ShapeDtypeStruct((N, H), jnp.float32),
    )(x, glove, W1, degp)


def _tc_mid(ap, g, b, W, degp):
    N, F = g.shape
    H2 = W.shape[1]
    grid = (pl.cdiv(N, _BN),)
    return pl.pallas_call(
        _tc_mid_body,
        grid=grid,
        in_specs=[
            pl.BlockSpec((2, _BN, F), lambda i: (0, i, 0)),
            pl.BlockSpec((_BN, F), lambda i: (i, 0)),
            pl.BlockSpec((1, F), lambda i: (0, 0)),
            pl.BlockSpec((F, H2), lambda i: (0, 0)),
            pl.BlockSpec((2, _BN, 16), lambda i: (0, i, 0)),
        ],
        out_specs=pl.BlockSpec((_BN, H2), lambda i: (i, 0)),
        out_shape=jax.ShapeDtypeStruct((N, H2), jnp.float32),
    )(ap, g, b, W, degp)


def _tc_final(ap, g, b, degp):
    N, F = g.shape
    grid = (pl.cdiv(N, _BN),)
    return pl.pallas_call(
        _tc_final_body,
        grid=grid,
        in_specs=[
            pl.BlockSpec((2, _BN, F), lambda i: (0, i, 0)),
            pl.BlockSpec((_BN, F), lambda i: (i, 0)),
            pl.BlockSpec((1, F), lambda i: (0, 0)),
            pl.BlockSpec((2, _BN, 16), lambda i: (0, i, 0)),
        ],
        out_specs=pl.BlockSpec((_BN, F), lambda i: (i, 0)),
        out_shape=jax.ShapeDtypeStruct((N, F), jnp.float32),
    )(ap, g, b, degp)


def kernel(x, edge_index, glove, W1, b1, W2, b2, W3, b3):
    N, D = x.shape
    E = edge_index.shape[1]
    H = W1.shape[1]
    C = W3.shape[1]

    NP = (-(-(N + 1) // 128)) * 128

    NB = 2 * PF
    pair = -(-(-(-E // K)) // NS)
    pair = (-(-pair // NB)) * NB
    CH1 = max(NB, int(round(pair * 0.25 / NB)) * NB)
    CH0 = pair - CH1
    R = NS * pair + abs(CH0 - CH1)
    pad = R * K - E

    src = jnp.concatenate(
        [edge_index[0], jnp.zeros((pad,), jnp.int32)]).reshape(R, K)
    dst = jnp.concatenate(
        [edge_index[1], jnp.full((pad,), N, jnp.int32)]).reshape(R, K)

    deg_k = _make_deg_kernel(NP, CH0, CH1)
    agg_h = _make_agg_kernel(NP, CH0, CH1, H)
    agg_c = _make_agg_kernel(NP, CH0, CH1, C)

    degp = deg_k(dst)
    g1 = _tc_first(x, glove, W1, degp)
    a1 = agg_h(g1, src, dst)
    g2 = _tc_mid(a1, g1, b1.reshape(1, H), W2, degp)
    a2 = agg_h(g2, src, dst)
    g3 = _tc_mid(a2, g2, b2.reshape(1, H), W3, degp)
    a3 = agg_c(g3, src, dst)
    return _tc_final(a3, g3, b3.reshape(1, C), degp)


# speedup vs baseline: 1.2288x; 1.0346x over previous
import functools

import jax
import jax.numpy as jnp
from jax import lax
from jax.experimental import pallas as pl
from jax.experimental.pallas import tpu as pltpu
from jax.experimental.pallas import tpu_sc as plsc

NC = 2
NS = 16
NW = NC * NS
K = 128


def _zero_rows(ref, nrows, width):
    def body(i, _):
        for j in range(width // 16):
            ref[i, pl.ds(j * 16, 16)] = jnp.zeros((16,), jnp.float32)
        return 0
    lax.fori_loop(0, nrows, body, 0)


def _chunk_base(c, s, CH0, CH1):
    base = lax.select(c == 0, s * CH0, 16 * CH0 + s * CH1)
    nch = lax.select(c == 0, CH0, CH1)
    return base, nch


def _make_deg_kernel(NP, CH0, CH1):
    RPT = NP // NS
    CHM = max(CH0, CH1)
    Fd = 16
    mesh = plsc.VectorSubcoreMesh(core_axis_name="c", subcore_axis_name="s")

    @functools.partial(
        pl.kernel,
        out_type=jax.ShapeDtypeStruct((NC, NP, Fd), jnp.float32),
        mesh=mesh,
        scratch_types=[
            pltpu.VMEM((CHM, K), jnp.int32),
            pltpu.VMEM((K, Fd), jnp.float32),
            pltpu.VMEM((RPT, Fd), jnp.float32),
            pltpu.VMEM_SHARED((NP, Fd), jnp.float32),
            pltpu.SemaphoreType.DMA,
        ],
        compiler_params=pltpu.CompilerParams(use_tc_tiling_on_sc=False),
    )
    def deg_kernel(dst_hbm, out_hbm, dst_v, ones_v, zbuf_v, acc_sh, sem_s):
        c = lax.axis_index("c")
        s = lax.axis_index("s")
        base, nch = _chunk_base(c, s, CH0, CH1)

        pltpu.sync_copy(dst_hbm.at[pl.ds(base, CHM)], dst_v)

        def fill_ones(i, _):
            ones_v[i, pl.ds(0, 16)] = jnp.full((16,), 1.0, jnp.float32)
            return 0
        lax.fori_loop(0, K, fill_ones, 0)
        _zero_rows(zbuf_v, RPT, Fd)

        pltpu.sync_copy(zbuf_v, acc_sh.at[pl.ds(s * RPT, RPT)])
        plsc.subcore_barrier()

        def step(j, _):
            pltpu.async_copy(ones_v, acc_sh.at[dst_v.at[j]], sem_s, add=True)

            @pl.when(j >= 8)
            def _():
                pltpu.make_async_copy(
                    ones_v, acc_sh.at[dst_v.at[j]], sem_s).wait()
            return 0
        lax.fori_loop(0, nch, step, 0)

        def drain(j, _):
            pltpu.make_async_copy(ones_v, acc_sh.at[dst_v.at[j]], sem_s).wait()
            return 0
        lax.fori_loop(0, 8, drain, 0)

        plsc.subcore_barrier()
        pltpu.sync_copy(acc_sh.at[pl.ds(s * RPT, RPT)], zbuf_v)
        pltpu.sync_copy(zbuf_v, out_hbm.at[c, pl.ds(s * RPT, RPT)])

    return deg_kernel


PF = 4


def _make_agg_kernel(NP, CH0, CH1, F):
    RPT = NP // NS
    CHM = max(CH0, CH1)
    mesh = plsc.VectorSubcoreMesh(core_axis_name="c", subcore_axis_name="s")

    NB = 2 * PF
    assert CH0 % NB == 0 and CH0 >= NB
    assert CH1 % NB == 0 and CH1 >= NB

    @functools.partial(
        pl.kernel,
        out_type=jax.ShapeDtypeStruct((NC, NP, F), jnp.float32),
        mesh=mesh,
        scratch_types=[
            pltpu.VMEM((CHM, K), jnp.int32),
            pltpu.VMEM((CHM, K), jnp.int32),
            pltpu.VMEM((NB, K, F), jnp.float32),
            pltpu.VMEM((RPT, F), jnp.float32),
            pltpu.VMEM_SHARED((NP, F), jnp.float32),
        ] + [pltpu.SemaphoreType.DMA] * (2 * NB),
        compiler_params=pltpu.CompilerParams(use_tc_tiling_on_sc=False),
    )
    def agg_kernel(g_hbm, src_hbm, dst_hbm, out_hbm,
                   src_v, dst_v, rows_v, zbuf_v, acc_sh, *sems):
        sem_g = sems[:NB]
        sem_s = sems[NB:]
        c = lax.axis_index("c")
        s = lax.axis_index("s")
        base, nch = _chunk_base(c, s, CH0, CH1)

        pltpu.sync_copy(src_hbm.at[pl.ds(base, CHM)], src_v)
        pltpu.sync_copy(dst_hbm.at[pl.ds(base, CHM)], dst_v)
        _zero_rows(zbuf_v, RPT, F)

        pltpu.sync_copy(zbuf_v, acc_sh.at[pl.ds(s * RPT, RPT)])
        plsc.subcore_barrier()

        def gather(ck, b):
            return pltpu.make_async_copy(
                g_hbm.at[src_v.at[ck]], rows_v.at[b], sem_g[b])

        def scatter(ck, b):
            return pltpu.make_async_copy(
                rows_v.at[b], acc_sh.at[dst_v.at[ck]], sem_s[b])

        for ck in range(PF):
            gather(ck, ck % NB).start()

        def group(j, _):
            c0 = j * NB
            for b in range(NB):
                ck = c0 + b
                gather(ck, b).wait()
                scatter(ck, b).start(add=True)
                pre = ck + PF
                bp = (b + PF) % NB

                @pl.when(pre < nch)
                def _():
                    @pl.when(pre >= NB)
                    def _():
                        scatter(pre - NB, bp).wait()
                    gather(pre, bp).start()
            return 0
        lax.fori_loop(0, nch // NB, group, 0)

        for b in range(NB):
            scatter(nch - NB + b, b).wait()

        plsc.subcore_barrier()
        pltpu.sync_copy(acc_sh.at[pl.ds(s * RPT, RPT)], zbuf_v)
        pltpu.sync_copy(zbuf_v, out_hbm.at[c, pl.ds(s * RPT, RPT)])

    return agg_kernel


def _dinv_block(dp):
    deg = dp[0, :, 0:1] + dp[1, :, 0:1] + 1.0
    return lax.rsqrt(jnp.maximum(deg, 1.0))


def _tc_first_body(x_ref, gl_ref, w_ref, dp_ref, o_ref):
    dinv = _dinv_block(dp_ref[...])
    h0 = jnp.dot(x_ref[...], gl_ref[...], preferred_element_type=jnp.float32)
    o_ref[...] = jnp.dot(h0, w_ref[...], preferred_element_type=jnp.float32) * dinv


def _tc_mid_body(ap_ref, g_ref, b_ref, w_ref, dp_ref, o_ref):
    dinv = _dinv_block(dp_ref[...])
    pre = ap_ref[0] + ap_ref[1] + g_ref[...]
    h = jnp.maximum(pre * dinv + b_ref[...], 0.0)
    o_ref[...] = jnp.dot(h, w_ref[...], preferred_element_type=jnp.float32) * dinv


def _tc_final_body(ap_ref, g_ref, b_ref, dp_ref, o_ref):
    dinv = _dinv_block(dp_ref[...])
    h = (ap_ref[0] + ap_ref[1] + g_ref[...]) * dinv + b_ref[...]
    m = jnp.max(h, axis=1, keepdims=True)
    z = h - m
    lse = jnp.log(jnp.sum(jnp.exp(z), axis=1, keepdims=True))
    o_ref[...] = z - lse


_BN = 1024


def _tc_first(x, glove, W1, degp):
    N, D = x.shape
    H = W1.shape[1]
    grid = (pl.cdiv(N, _BN),)
    return pl.pallas_call(
        _tc_first_body,
        grid=grid,
        in_specs=[
            pl.BlockSpec((_BN, D), lambda i: (i, 0)),
            pl.BlockSpec((D, D), lambda i: (0, 0)),
            pl.BlockSpec((D, H), lambda i: (0, 0)),
            pl.BlockSpec((2, _BN, 16), lambda i: (0, i, 0)),
        ],
        out_specs=pl.BlockSpec((_BN, H), lambda i: (i, 0)),
        out_shape=jax.ShapeDtypeStruct((N, H), jnp.float32),
    )(x, glove, W1, degp)


def _tc_mid(ap, g, b, W, degp):
    N, F = g.shape
    H2 = W.shape[1]
    grid = (pl.cdiv(N, _BN),)
    return pl.pallas_call(
        _tc_mid_body,
        grid=grid,
        in_specs=[
            pl.BlockSpec((2, _BN, F), lambda i: (0, i, 0)),
            pl.BlockSpec((_BN, F), lambda i: (i, 0)),
            pl.BlockSpec((1, F), lambda i: (0, 0)),
            pl.BlockSpec((F, H2), lambda i: (0, 0)),
            pl.BlockSpec((2, _BN, 16), lambda i: (0, i, 0)),
        ],
        out_specs=pl.BlockSpec((_BN, H2), lambda i: (i, 0)),
        out_shape=jax.ShapeDtypeStruct((N, H2), jnp.float32),
    )(ap, g, b, W, degp)


def _tc_final(ap, g, b, degp):
    N, F = g.shape
    grid = (pl.cdiv(N, _BN),)
    return pl.pallas_call(
        _tc_final_body,
        grid=grid,
        in_specs=[
            pl.BlockSpec((2, _BN, F), lambda i: (0, i, 0)),
            pl.BlockSpec((_BN, F), lambda i: (i, 0)),
            pl.BlockSpec((1, F), lambda i: (0, 0)),
            pl.BlockSpec((2, _BN, 16), lambda i: (0, i, 0)),
        ],
        out_specs=pl.BlockSpec((_BN, F), lambda i: (i, 0)),
        out_shape=jax.ShapeDtypeStruct((N, F), jnp.float32),
    )(ap, g, b, degp)


def kernel(x, edge_index, glove, W1, b1, W2, b2, W3, b3):
    N, D = x.shape
    E = edge_index.shape[1]
    H = W1.shape[1]
    C = W3.shape[1]

    NP = (-(-(N + 1) // 128)) * 128

    NB = 2 * PF
    pair = -(-(-(-E // K)) // NS)
    pair = (-(-pair // NB)) * NB
    CH1 = NB
    CH0 = pair - CH1
    R = NS * pair + abs(CH0 - CH1)
    pad = R * K - E

    src = jnp.concatenate(
        [edge_index[0], jnp.zeros((pad,), jnp.int32)]).reshape(R, K)
    dst = jnp.concatenate(
        [edge_index[1], jnp.full((pad,), N, jnp.int32)]).reshape(R, K)

    deg_k = _make_deg_kernel(NP, CH0, CH1)
    agg_h = _make_agg_kernel(NP, CH0, CH1, H)
    agg_c = _make_agg_kernel(NP, CH0, CH1, C)

    degp = deg_k(dst)
    g1 = _tc_first(x, glove, W1, degp)
    a1 = agg_h(g1, src, dst)
    g2 = _tc_mid(a1, g1, b1.reshape(1, H), W2, degp)
    a2 = agg_h(g2, src, dst)
    g3 = _tc_mid(a2, g2, b2.reshape(1, H), W3, degp)
    a3 = agg_c(g3, src, dst)
    return _tc_final(a3, g3, b3.reshape(1, C), degp)


# speedup vs baseline: 1.2504x; 1.0176x over previous
import functools

import jax
import jax.numpy as jnp
from jax import lax
from jax.experimental import pallas as pl
from jax.experimental.pallas import tpu as pltpu
from jax.experimental.pallas import tpu_sc as plsc

NC = 2
NS = 16
NW = NC * NS
K = 128


def _zero_rows(ref, nrows, width):
    def body(i, _):
        for j in range(width // 16):
            ref[i, pl.ds(j * 16, 16)] = jnp.zeros((16,), jnp.float32)
        return 0
    lax.fori_loop(0, nrows, body, 0)


def _chunk_base(c, s, CH0, CH1):
    base = lax.select(c == 0, s * CH0, 16 * CH0 + s * CH1)
    nch = lax.select(c == 0, CH0, CH1)
    return base, nch


def _make_deg_kernel(NP, CH0, CH1):
    RPT = NP // NS
    CHM = max(CH0, CH1)
    Fd = 16
    mesh = plsc.VectorSubcoreMesh(core_axis_name="c", subcore_axis_name="s")

    @functools.partial(
        pl.kernel,
        out_type=jax.ShapeDtypeStruct((NC, NP, Fd), jnp.float32),
        mesh=mesh,
        scratch_types=[
            pltpu.VMEM((CHM, K), jnp.int32),
            pltpu.VMEM((K, Fd), jnp.float32),
            pltpu.VMEM((RPT, Fd), jnp.float32),
            pltpu.VMEM_SHARED((NP, Fd), jnp.float32),
            pltpu.SemaphoreType.DMA,
        ],
        compiler_params=pltpu.CompilerParams(use_tc_tiling_on_sc=False),
    )
    def deg_kernel(dst_hbm, out_hbm, dst_v, ones_v, zbuf_v, acc_sh, sem_s):
        c = lax.axis_index("c")
        s = lax.axis_index("s")
        base, nch = _chunk_base(c, s, CH0, CH1)

        pltpu.sync_copy(dst_hbm.at[pl.ds(base, CHM)], dst_v)

        def fill_ones(i, _):
            ones_v[i, pl.ds(0, 16)] = jnp.full((16,), 1.0, jnp.float32)
            return 0
        lax.fori_loop(0, K, fill_ones, 0)
        _zero_rows(zbuf_v, RPT, Fd)

        pltpu.sync_copy(zbuf_v, acc_sh.at[pl.ds(s * RPT, RPT)])
        plsc.subcore_barrier()

        def step(j, _):
            pltpu.async_copy(ones_v, acc_sh.at[dst_v.at[j]], sem_s, add=True)

            @pl.when(j >= 8)
            def _():
                pltpu.make_async_copy(
                    ones_v, acc_sh.at[dst_v.at[j]], sem_s).wait()
            return 0
        lax.fori_loop(0, nch, step, 0)

        def drain(j, _):
            pltpu.make_async_copy(ones_v, acc_sh.at[dst_v.at[j]], sem_s).wait()
            return 0
        lax.fori_loop(0, 8, drain, 0)

        plsc.subcore_barrier()
        pltpu.sync_copy(acc_sh.at[pl.ds(s * RPT, RPT)], zbuf_v)
        pltpu.sync_copy(zbuf_v, out_hbm.at[c, pl.ds(s * RPT, RPT)])

    return deg_kernel


PF = 4


def _make_agg_kernel(NP, CH0, CH1, F):
    RPT = NP // NS
    CHM = max(CH0, CH1)
    mesh = plsc.VectorSubcoreMesh(core_axis_name="c", subcore_axis_name="s")

    NB = 2 * PF
    assert CH0 % NB == 0 and CH0 >= NB
    assert CH1 % NB == 0 and CH1 >= NB

    @functools.partial(
        pl.kernel,
        out_type=jax.ShapeDtypeStruct((NC, NP, F), jnp.float32),
        mesh=mesh,
        scratch_types=[
            pltpu.VMEM((CHM, K), jnp.int32),
            pltpu.VMEM((CHM, K), jnp.int32),
            pltpu.VMEM((NB, K, F), jnp.float32),
            pltpu.VMEM((RPT, F), jnp.float32),
            pltpu.VMEM_SHARED((NP, F), jnp.float32),
        ] + [pltpu.SemaphoreType.DMA] * (2 * NB),
        compiler_params=pltpu.CompilerParams(use_tc_tiling_on_sc=False),
    )
    def agg_kernel(g_hbm, src_hbm, dst_hbm, out_hbm,
                   src_v, dst_v, rows_v, zbuf_v, acc_sh, *sems):
        sem_g = sems[:NB]
        sem_s = sems[NB:]
        c = lax.axis_index("c")
        s = lax.axis_index("s")
        base, nch = _chunk_base(c, s, CH0, CH1)

        lo = min(CH0, CH1)
        pltpu.sync_copy(src_hbm.at[pl.ds(base, lo)], src_v.at[pl.ds(0, lo)])
        pltpu.sync_copy(dst_hbm.at[pl.ds(base, lo)], dst_v.at[pl.ds(0, lo)])
        if CH0 != CH1:
            @pl.when(nch == CHM)
            def _():
                pltpu.sync_copy(src_hbm.at[pl.ds(base + lo, CHM - lo)],
                                src_v.at[pl.ds(lo, CHM - lo)])
                pltpu.sync_copy(dst_hbm.at[pl.ds(base + lo, CHM - lo)],
                                dst_v.at[pl.ds(lo, CHM - lo)])
        _zero_rows(zbuf_v, RPT, F)

        pltpu.sync_copy(zbuf_v, acc_sh.at[pl.ds(s * RPT, RPT)])
        plsc.subcore_barrier()

        def gather(ck, b):
            return pltpu.make_async_copy(
                g_hbm.at[src_v.at[ck]], rows_v.at[b], sem_g[b])

        def scatter(ck, b):
            return pltpu.make_async_copy(
                rows_v.at[b], acc_sh.at[dst_v.at[ck]], sem_s[b])

        for ck in range(PF):
            gather(ck, ck % NB).start()

        def group(j, _):
            c0 = j * NB
            for b in range(NB):
                ck = c0 + b
                gather(ck, b).wait()
                scatter(ck, b).start(add=True)
                pre = ck + PF
                bp = (b + PF) % NB

                @pl.when(pre < nch)
                def _():
                    @pl.when(pre >= NB)
                    def _():
                        scatter(pre - NB, bp).wait()
                    gather(pre, bp).start()
            return 0
        lax.fori_loop(0, nch // NB, group, 0)

        for b in range(NB):
            scatter(nch - NB + b, b).wait()

        plsc.subcore_barrier()
        pltpu.sync_copy(acc_sh.at[pl.ds(s * RPT, RPT)], zbuf_v)
        pltpu.sync_copy(zbuf_v, out_hbm.at[c, pl.ds(s * RPT, RPT)])

    return agg_kernel


def _dinv_block(dp):
    deg = dp[0, :, 0:1] + dp[1, :, 0:1] + 1.0
    return lax.rsqrt(jnp.maximum(deg, 1.0))


def _tc_first_body(x_ref, gl_ref, w_ref, dp_ref, o_ref):
    dinv = _dinv_block(dp_ref[...])
    h0 = jnp.dot(x_ref[...], gl_ref[...], preferred_element_type=jnp.float32)
    o_ref[...] = jnp.dot(h0, w_ref[...], preferred_element_type=jnp.float32) * dinv


def _tc_mid_body(ap_ref, g_ref, b_ref, w_ref, dp_ref, o_ref):
    dinv = _dinv_block(dp_ref[...])
    pre = ap_ref[0] + ap_ref[1] + g_ref[...]
    h = jnp.maximum(pre * dinv + b_ref[...], 0.0)
    o_ref[...] = jnp.dot(h, w_ref[...], preferred_element_type=jnp.float32) * dinv


def _tc_final_body(ap_ref, g_ref, b_ref, dp_ref, o_ref):
    dinv = _dinv_block(dp_ref[...])
    h = (ap_ref[0] + ap_ref[1] + g_ref[...]) * dinv + b_ref[...]
    m = jnp.max(h, axis=1, keepdims=True)
    z = h - m
    lse = jnp.log(jnp.sum(jnp.exp(z), axis=1, keepdims=True))
    o_ref[...] = z - lse


_BN = 1024


def _tc_first(x, glove, W1, degp):
    N, D = x.shape
    H = W1.shape[1]
    grid = (pl.cdiv(N, _BN),)
    return pl.pallas_call(
        _tc_first_body,
        grid=grid,
        in_specs=[
            pl.BlockSpec((_BN, D), lambda i: (i, 0)),
            pl.BlockSpec((D, D), lambda i: (0, 0)),
            pl.BlockSpec((D, H), lambda i: (0, 0)),
            pl.BlockSpec((2, _BN, 16), lambda i: (0, i, 0)),
        ],
        out_specs=pl.BlockSpec((_BN, H), lambda i: (i, 0)),
        out_shape=jax.ShapeDtypeStruct((N, H), jnp.float32),
    )(x, glove, W1, degp)


def _tc_mid(ap, g, b, W, degp):
    N, F = g.shape
    H2 = W.shape[1]
    grid = (pl.cdiv(N, _BN),)
    return pl.pallas_call(
        _tc_mid_body,
        grid=grid,
        in_specs=[
            pl.BlockSpec((2, _BN, F), lambda i: (0, i, 0)),
            pl.BlockSpec((_BN, F), lambda i: (i, 0)),
            pl.BlockSpec((1, F), lambda i: (0, 0)),
            pl.BlockSpec((F, H2), lambda i: (0, 0)),
            pl.BlockSpec((2, _BN, 16), lambda i: (0, i, 0)),
        ],
        out_specs=pl.BlockSpec((_BN, H2), lambda i: (i, 0)),
        out_shape=jax.ShapeDtypeStruct((N, H2), jnp.float32),
    )(ap, g, b, W, degp)


def _tc_final(ap, g, b, degp):
    N, F = g.shape
    grid = (pl.cdiv(N, _BN),)
    return pl.pallas_call(
        _tc_final_body,
        grid=grid,
        in_specs=[
            pl.BlockSpec((2, _BN, F), lambda i: (0, i, 0)),
            pl.BlockSpec((_BN, F), lambda i: (i, 0)),
            pl.BlockSpec((1, F), lambda i: (0, 0)),
            pl.BlockSpec((2, _BN, 16), lambda i: (0, i, 0)),
        ],
        out_specs=pl.BlockSpec((_BN, F), lambda i: (i, 0)),
        out_shape=jax.ShapeDtypeStruct((N, F), jnp.float32),
    )(ap, g, b, degp)


def kernel(x, edge_index, glove, W1, b1, W2, b2, W3, b3):
    N, D = x.shape
    E = edge_index.shape[1]
    H = W1.shape[1]
    C = W3.shape[1]

    NP = (-(-(N + 1) // 128)) * 128

    NB = 2 * PF
    pair = -(-(-(-E // K)) // NS)
    pair = (-(-pair // NB)) * NB
    CH1 = NB
    CH0 = pair - CH1
    R = NS * pair + abs(CH0 - CH1)
    pad = R * K - E

    src = jnp.concatenate(
        [edge_index[0], jnp.zeros((pad,), jnp.int32)]).reshape(R, K)
    dst = jnp.concatenate(
        [edge_index[1], jnp.full((pad,), N, jnp.int32)]).reshape(R, K)

    deg_k = _make_deg_kernel(NP, CH0, CH1)
    agg_h = _make_agg_kernel(NP, CH0, CH1, H)
    agg_c = _make_agg_kernel(NP, CH0, CH1, C)

    degp = deg_k(dst)
    g1 = _tc_first(x, glove, W1, degp)
    a1 = agg_h(g1, src, dst)
    g2 = _tc_mid(a1, g1, b1.reshape(1, H), W2, degp)
    a2 = agg_h(g2, src, dst)
    g3 = _tc_mid(a2, g2, b2.reshape(1, H), W3, degp)
    a3 = agg_c(g3, src, dst)
    return _tc_final(a3, g3, b3.reshape(1, C), degp)
